# Initial kernel scaffold; baseline (speedup 1.0000x reference)
import jax
import jax.numpy as jnp
from jax.experimental import pallas as pl


def kernel(x, edge_index, batch, W1, b1, W2, b2, W3, b3, Wm1, bm1, Wm2, bm2):
    raise NotImplementedError("write your pallas kernel here")


# speedup vs baseline: 18.9649x; 18.9649x over previous
import functools

import jax
import jax.numpy as jnp
from jax import lax
from jax.experimental import pallas as pl
from jax.experimental.pallas import tpu as pltpu
from jax.experimental.pallas import tpu_sc as plsc

N = 10000
E = 320000
G = 16
D_IN = 128
H = 64
NUM_CLASSES = 10

EW = 128
E_PAD = 327680
ROWS = E_PAD // EW
SUP = 16
NB = 2000

NC, NS = 2, 16
N_OUT = 10240
N_ACC = N_OUT + 16
STRIPE = N_OUT // NS
WCH = 128


@functools.lru_cache(maxsize=None)
def _mesh():
    return plsc.VectorSubcoreMesh(core_axis_name="c", subcore_axis_name="s")


def _zero_stripe(zeros_hbm, zbuf, acc, s):
    pltpu.sync_copy(zeros_hbm, zbuf)
    for t in range(STRIPE // WCH):
        pltpu.sync_copy(zbuf, acc.at[pl.ds(s * STRIPE + t * WCH, WCH)])


def _writeback_stripe(acc, zbuf, out_hbm, c, s):
    for t in range(STRIPE // WCH):
        pltpu.sync_copy(acc.at[pl.ds(s * STRIPE + t * WCH, WCH)], zbuf)
        pltpu.sync_copy(zbuf, out_hbm.at[c].at[pl.ds(s * STRIPE + t * WCH, WCH)])


def _sc_degree_body(dst_hbm, ones_hbm, zeros_hbm, out_hbm, idx_v, ones_v, zbuf, acc):
    c = lax.axis_index("c")
    s = lax.axis_index("s")
    pltpu.sync_copy(ones_hbm, ones_v)
    _zero_stripe(zeros_hbm, zbuf, acc, s)
    plsc.subcore_barrier()

    rps = ROWS // (NC * NS)
    row0 = c * (ROWS // NC) + s * rps

    def super_body(sup, _):
        pltpu.sync_copy(dst_hbm.at[pl.ds(row0 + sup * SUP, SUP)], idx_v)

        def edge_body(j, _):
            pltpu.sync_copy(ones_v, acc.at[idx_v.at[j]], add=True)
            return 0

        lax.fori_loop(0, SUP, edge_body, 0)
        return 0

    lax.fori_loop(0, rps // SUP, super_body, 0)
    plsc.subcore_barrier()
    _writeback_stripe(acc, zbuf, out_hbm, c, s)


@functools.lru_cache(maxsize=None)
def _sc_degree_kernel():
    return pl.kernel(
        _sc_degree_body,
        out_type=jax.ShapeDtypeStruct((NC, N_OUT, 16), jnp.float32),
        mesh=_mesh(),
        scratch_types=[
            pltpu.VMEM((SUP, EW), jnp.int32),
            pltpu.VMEM((EW, 16), jnp.float32),
            pltpu.VMEM((WCH, 16), jnp.float32),
            pltpu.VMEM_SHARED((N_ACC, 16), jnp.float32),
        ],
    )


def _sc_degree(dst2d, ones16, zeros16):
    return _sc_degree_kernel()(dst2d, ones16, zeros16)


def _make_sc_propagate(T):
    def body(tbl_hbm, src_hbm, dst_hbm, zeros_hbm, out_hbm,
             srci, dsti, rows_v, zbuf, acc, sem):
        c = lax.axis_index("c")
        s = lax.axis_index("s")
        _zero_stripe(zeros_hbm, zbuf, acc, s)
        plsc.subcore_barrier()

        if T == 1:
            tbl = tbl_hbm.at[0]
            rps = ROWS // (NC * NS)
            row0 = c * (ROWS // NC) + s * rps
        else:
            tbl = tbl_hbm.at[c]
            rps = ROWS // NS
            row0 = s * rps

        def super_body(sup, _):
            base = row0 + sup * SUP
            pltpu.sync_copy(src_hbm.at[pl.ds(base, SUP)], srci)
            pltpu.sync_copy(dst_hbm.at[pl.ds(base, SUP)], dsti)

            def edge_body(j, _):
                pltpu.async_copy(tbl.at[srci.at[j]], rows_v, sem).wait()
                pltpu.sync_copy(rows_v, acc.at[dsti.at[j]], add=True)
                return 0

            lax.fori_loop(0, SUP, edge_body, 0)
            return 0

        lax.fori_loop(0, rps // SUP, super_body, 0)
        plsc.subcore_barrier()
        _writeback_stripe(acc, zbuf, out_hbm, c, s)

    @functools.lru_cache(maxsize=None)
    def build():
        return pl.kernel(
            body,
            out_type=jax.ShapeDtypeStruct((NC, N_OUT, 128), jnp.float32),
            mesh=_mesh(),
            scratch_types=[
                pltpu.VMEM((SUP, EW), jnp.int32),
                pltpu.VMEM((SUP, EW), jnp.int32),
                pltpu.VMEM((EW, 128), jnp.float32),
                pltpu.VMEM((WCH, 128), jnp.float32),
                pltpu.VMEM_SHARED((N_ACC, 128), jnp.float32),
                pltpu.SemaphoreType.DMA,
            ],
        )

    def call(table, src2d, dst2d, zeros):
        return build()(table, src2d, dst2d, zeros)

    return call


_sc_propagate_split_edges = _make_sc_propagate(1)
_sc_propagate_split_feats = _make_sc_propagate(2)


def _dv(degp):
    total = degp[0, :, 0:1] + degp[1, :, 0:1]
    return lax.rsqrt(total + 1.0)


def _t1_body(x_ref, w1_ref, degp_ref, out_ref):
    dv = _dv(degp_ref[...])
    u = jnp.dot(x_ref[...], w1_ref[...], preferred_element_type=jnp.float32) * dv
    out_ref[0] = jnp.concatenate([u, jnp.zeros((NB, 128 - H), jnp.float32)], axis=1)


def _tc_t1(x, W1, degp):
    return pl.pallas_call(
        _t1_body,
        grid=(N // NB,),
        in_specs=[
            pl.BlockSpec((NB, D_IN), lambda i: (i, 0)),
            pl.BlockSpec((D_IN, H), lambda i: (0, 0)),
            pl.BlockSpec((2, NB, 16), lambda i: (0, i, 0)),
        ],
        out_specs=pl.BlockSpec((1, NB, 128), lambda i: (0, i, 0)),
        out_shape=jax.ShapeDtypeStruct((1, N, 128), jnp.float32),
    )(x, W1, degp)


def _t2_body(s1_ref, t1_ref, degp_ref, b1_ref, out_ref):
    dv = _dv(degp_ref[...])
    u = dv * (s1_ref[0] + s1_ref[1] + t1_ref[0])
    h1 = jnp.maximum(u[:, :H] + b1_ref[...][0][None, :], 0.0) * dv
    out_ref[0] = jnp.concatenate([h1, jnp.zeros((NB, 128 - H), jnp.float32)], axis=1)


def _tc_t2(S1, t1, degp, b1):
    return pl.pallas_call(
        _t2_body,
        grid=(N // NB,),
        in_specs=[
            pl.BlockSpec((2, NB, 128), lambda i: (0, i, 0)),
            pl.BlockSpec((1, NB, 128), lambda i: (0, i, 0)),
            pl.BlockSpec((2, NB, 16), lambda i: (0, i, 0)),
            pl.BlockSpec((1, H), lambda i: (0, 0)),
        ],
        out_specs=pl.BlockSpec((1, NB, 128), lambda i: (0, i, 0)),
        out_shape=jax.ShapeDtypeStruct((1, N, 128), jnp.float32),
    )(S1, t1, degp, b1.reshape(1, H))


def _t3_body(s2_ref, t2_ref, degp_ref, w2_ref, b2_ref, out_ref):
    dv = _dv(degp_ref[...])
    u = dv * (s2_ref[0] + s2_ref[1] + t2_ref[0])
    h2 = jnp.dot(u[:, :H], w2_ref[...], preferred_element_type=jnp.float32)
    h2 = jnp.maximum(h2 + b2_ref[...][0][None, :], 0.0) * dv
    out_ref[0] = h2[:, :128]
    out_ref[1] = h2[:, 128:]


def _tc_t3(S2, t2, degp, W2, b2):
    return pl.pallas_call(
        _t3_body,
        grid=(N // NB,),
        in_specs=[
            pl.BlockSpec((2, NB, 128), lambda i: (0, i, 0)),
            pl.BlockSpec((1, NB, 128), lambda i: (0, i, 0)),
            pl.BlockSpec((2, NB, 16), lambda i: (0, i, 0)),
            pl.BlockSpec((H, 4 * H), lambda i: (0, 0)),
            pl.BlockSpec((1, 4 * H), lambda i: (0, 0)),
        ],
        out_specs=pl.BlockSpec((2, NB, 128), lambda i: (0, i, 0)),
        out_shape=jax.ShapeDtypeStruct((2, N, 128), jnp.float32),
    )(S2, t2, degp, W2, b2.reshape(1, 4 * H))


def _pool_body(s3_ref, t3_ref, degp_ref, w3_ref, b3_ref, batch_ref,
               psum_ref, pmax_ref):
    dv = _dv(degp_ref[...])
    acc = jnp.zeros((NB, 8 * H), jnp.float32)
    for c in range(2):
        u = dv * (s3_ref[c] + t3_ref[c])
        acc = acc + jnp.dot(u, w3_ref[c], preferred_element_type=jnp.float32)
    h3 = jnp.maximum(acc + b3_ref[...][0][None, :], 0.0)

    @pl.when(pl.program_id(0) == 0)
    def _():
        psum_ref[...] = jnp.zeros_like(psum_ref)
        pmax_ref[...] = jnp.full_like(pmax_ref, -jnp.inf)

    b = batch_ref[...]
    for g in range(G):
        mask = b == g
        psum_ref[g, :] += jnp.sum(jnp.where(mask, h3, 0.0), axis=0)
        pmax_ref[g, :] = jnp.maximum(
            pmax_ref[g, :], jnp.max(jnp.where(mask, h3, -jnp.inf), axis=0))


def _tc_pool(S3, t3, degp, W3r, b3, batch2d):
    return pl.pallas_call(
        _pool_body,
        grid=(N // NB,),
        in_specs=[
            pl.BlockSpec((2, NB, 128), lambda i: (0, i, 0)),
            pl.BlockSpec((2, NB, 128), lambda i: (0, i, 0)),
            pl.BlockSpec((2, NB, 16), lambda i: (0, i, 0)),
            pl.BlockSpec((2, 128, 8 * H), lambda i: (0, 0, 0)),
            pl.BlockSpec((1, 8 * H), lambda i: (0, 0)),
            pl.BlockSpec((NB, 1), lambda i: (i, 0)),
        ],
        out_specs=[
            pl.BlockSpec((G, 8 * H), lambda i: (0, 0)),
            pl.BlockSpec((G, 8 * H), lambda i: (0, 0)),
        ],
        out_shape=[
            jax.ShapeDtypeStruct((G, 8 * H), jnp.float32),
            jax.ShapeDtypeStruct((G, 8 * H), jnp.float32),
        ],
        compiler_params=pltpu.CompilerParams(
            dimension_semantics=("arbitrary",)),
    )(S3, t3, degp, W3r, b3.reshape(1, 8 * H), batch2d)


def _head_body(psum_ref, pmax_ref, batch_ref, wm1_ref, bm1_ref, wm2_ref,
               bm2_ref, out_ref):
    b = batch_ref[...]
    iota16 = lax.broadcasted_iota(jnp.int32, (G, 1), 0)
    inv = jnp.zeros((G, 1), jnp.float32)
    for g in range(G):
        cnt = jnp.sum(jnp.where(b == g, 1.0, 0.0))
        inv = inv + jnp.where(iota16 == g, 1.0 / jnp.maximum(cnt, 1.0), 0.0)
    mean = psum_ref[...] * inv
    hid = (jnp.dot(mean, wm1_ref[0], preferred_element_type=jnp.float32)
           + jnp.dot(pmax_ref[...], wm1_ref[1], preferred_element_type=jnp.float32)
           + bm1_ref[...][0][None, :])
    hid = jnp.maximum(hid, 0.0)
    out_ref[...] = (jnp.dot(hid, wm2_ref[...], preferred_element_type=jnp.float32)
                    + bm2_ref[...][0][None, :])


def _tc_head(psum, pmax, batch2d, Wm1r, bm1, Wm2, bm2):
    return pl.pallas_call(
        _head_body,
        out_shape=jax.ShapeDtypeStruct((G, NUM_CLASSES), jnp.float32),
    )(psum, pmax, batch2d, Wm1r, bm1.reshape(1, 8 * H), Wm2,
      bm2.reshape(1, NUM_CLASSES))


def kernel(x, edge_index, batch, W1, b1, W2, b2, W3, b3, Wm1, bm1, Wm2, bm2):
    pad = E_PAD - E
    pad_src = (jnp.arange(pad, dtype=jnp.int32) * 97) % N
    pad_dst = N_OUT + (jnp.arange(pad, dtype=jnp.int32) % 16)
    src2d = jnp.concatenate([edge_index[0], pad_src]).reshape(ROWS, EW)
    dst2d = jnp.concatenate([edge_index[1], pad_dst]).reshape(ROWS, EW)
    batch2d = batch.reshape(N, 1)
    ones16 = jnp.ones((EW, 16), jnp.float32)
    zeros16 = jnp.zeros((WCH, 16), jnp.float32)
    zeros128 = jnp.zeros((WCH, 128), jnp.float32)

    degp = _sc_degree(dst2d, ones16, zeros16)

    t1 = _tc_t1(x, W1, degp)
    S1 = _sc_propagate_split_edges(t1, src2d, dst2d, zeros128)
    t2 = _tc_t2(S1, t1, degp, b1)
    S2 = _sc_propagate_split_edges(t2, src2d, dst2d, zeros128)
    t3 = _tc_t3(S2, t2, degp, W2, b2)
    S3 = _sc_propagate_split_feats(t3, src2d, dst2d, zeros128)
    psum, pmax = _tc_pool(S3, t3, degp, W3.reshape(2, 128, 8 * H), b3, batch2d)
    return _tc_head(psum, pmax, batch2d, Wm1.reshape(2, 8 * H, 8 * H), bm1,
                    Wm2, bm2)


# speedup vs baseline: 26.4121x; 1.3927x over previous
import functools

import jax
import jax.numpy as jnp
from jax import lax
from jax.experimental import pallas as pl
from jax.experimental.pallas import tpu as pltpu
from jax.experimental.pallas import tpu_sc as plsc

N = 10000
E = 320000
G = 16
D_IN = 128
H = 64
NUM_CLASSES = 10

EW = 128
E_PAD = 327680
ROWS = E_PAD // EW
SUP = 16
NB = 2000

NC, NS = 2, 16
N_OUT = 10240
N_ACC = N_OUT + 16
STRIPE = N_OUT // NS
WCH = 64
DEPTH = 4


@functools.lru_cache(maxsize=None)
def _mesh():
    return plsc.VectorSubcoreMesh(core_axis_name="c", subcore_axis_name="s")


def _zero_stripe(zeros_hbm, zbuf, acc, s):
    pltpu.sync_copy(zeros_hbm, zbuf)
    for t in range(STRIPE // WCH):
        pltpu.sync_copy(zbuf, acc.at[pl.ds(s * STRIPE + t * WCH, WCH)])


def _writeback_stripe(acc, zbuf, out_hbm, c, s):
    for t in range(STRIPE // WCH):
        pltpu.sync_copy(acc.at[pl.ds(s * STRIPE + t * WCH, WCH)], zbuf)
        pltpu.sync_copy(zbuf, out_hbm.at[c].at[pl.ds(s * STRIPE + t * WCH, WCH)])


def _sc_degree_body(dst_hbm, ones_hbm, zeros_hbm, out_hbm, idx_v, ones_v, zbuf, acc):
    c = lax.axis_index("c")
    s = lax.axis_index("s")
    pltpu.sync_copy(ones_hbm, ones_v)
    _zero_stripe(zeros_hbm, zbuf, acc, s)
    plsc.subcore_barrier()

    rps = ROWS // (NC * NS)
    row0 = c * (ROWS // NC) + s * rps

    def super_body(sup, _):
        pltpu.sync_copy(dst_hbm.at[pl.ds(row0 + sup * SUP, SUP)], idx_v)

        def edge_body(j, _):
            pltpu.sync_copy(ones_v, acc.at[idx_v.at[j]], add=True)
            return 0

        lax.fori_loop(0, SUP, edge_body, 0)
        return 0

    lax.fori_loop(0, rps // SUP, super_body, 0)
    plsc.subcore_barrier()
    _writeback_stripe(acc, zbuf, out_hbm, c, s)


@functools.lru_cache(maxsize=None)
def _sc_degree_kernel():
    return pl.kernel(
        _sc_degree_body,
        out_type=jax.ShapeDtypeStruct((NC, N_OUT, 16), jnp.float32),
        mesh=_mesh(),
        scratch_types=[
            pltpu.VMEM((SUP, EW), jnp.int32),
            pltpu.VMEM((EW, 16), jnp.float32),
            pltpu.VMEM((WCH, 16), jnp.float32),
            pltpu.VMEM_SHARED((N_ACC, 16), jnp.float32),
        ],
    )


def _sc_degree(dst2d, ones16, zeros16):
    return _sc_degree_kernel()(dst2d, ones16, zeros16)


def _make_sc_propagate(T):
    rps = ROWS // (NC * NS) if T == 1 else ROWS // NS

    def body(tbl_hbm, src_hbm, dst_hbm, zeros_hbm, out_hbm,
             srci, dsti, b0, b1, zbuf, acc, s0, s1):
        c = lax.axis_index("c")
        s = lax.axis_index("s")
        _zero_stripe(zeros_hbm, zbuf, acc, s)
        plsc.subcore_barrier()

        if T == 1:
            tbl = tbl_hbm.at[0]
            row0 = c * (ROWS // NC) + s * rps
        else:
            tbl = tbl_hbm.at[c]
            row0 = s * rps

        def wait0():
            pltpu.make_async_copy(tbl.at[srci.at[0]], b0, s0).wait()

        def wait1():
            pltpu.make_async_copy(tbl.at[srci.at[0]], b1, s1).wait()

        def super_body(sup, _):
            base = row0 + sup * SUP
            pltpu.sync_copy(src_hbm.at[pl.ds(base, SUP)], srci)
            pltpu.sync_copy(dst_hbm.at[pl.ds(base, SUP)], dsti)
            pltpu.async_copy(tbl.at[srci.at[0]], b0, s0)
            pltpu.async_copy(tbl.at[srci.at[1]], b1, s1)

            def pair_body(k, _):
                o = 2 * k
                wait0()
                pltpu.sync_copy(b0, acc.at[dsti.at[o]], add=True)
                pltpu.async_copy(tbl.at[srci.at[o + 2]], b0, s0)
                wait1()
                pltpu.sync_copy(b1, acc.at[dsti.at[o + 1]], add=True)
                pltpu.async_copy(tbl.at[srci.at[o + 3]], b1, s1)
                return 0

            lax.fori_loop(0, SUP // 2 - 1, pair_body, 0)
            wait0()
            pltpu.sync_copy(b0, acc.at[dsti.at[SUP - 2]], add=True)
            wait1()
            pltpu.sync_copy(b1, acc.at[dsti.at[SUP - 1]], add=True)
            return 0

        lax.fori_loop(0, rps // SUP, super_body, 0)
        plsc.subcore_barrier()
        _writeback_stripe(acc, zbuf, out_hbm, c, s)

    @functools.lru_cache(maxsize=None)
    def build():
        return pl.kernel(
            body,
            out_type=jax.ShapeDtypeStruct((NC, N_OUT, 128), jnp.float32),
            mesh=_mesh(),
            scratch_types=[
                pltpu.VMEM((SUP, EW), jnp.int32),
                pltpu.VMEM((SUP, EW), jnp.int32),
                pltpu.VMEM((EW, 128), jnp.float32),
                pltpu.VMEM((EW, 128), jnp.float32),
                pltpu.VMEM((WCH, 128), jnp.float32),
                pltpu.VMEM_SHARED((N_ACC, 128), jnp.float32),
                pltpu.SemaphoreType.DMA,
                pltpu.SemaphoreType.DMA,
            ],
        )

    def call(table, src2d, dst2d, zeros):
        return build()(table, src2d, dst2d, zeros)

    return call


_sc_propagate_split_edges = _make_sc_propagate(1)
_sc_propagate_split_feats = _make_sc_propagate(2)


def _dv(degp):
    total = degp[0, :, 0:1] + degp[1, :, 0:1]
    return lax.rsqrt(total + 1.0)


def _t1_body(x_ref, w1_ref, degp_ref, out_ref):
    dv = _dv(degp_ref[...])
    u = jnp.dot(x_ref[...], w1_ref[...], preferred_element_type=jnp.float32) * dv
    out_ref[0] = jnp.concatenate([u, jnp.zeros((NB, 128 - H), jnp.float32)], axis=1)


def _tc_t1(x, W1, degp):
    return pl.pallas_call(
        _t1_body,
        grid=(N // NB,),
        in_specs=[
            pl.BlockSpec((NB, D_IN), lambda i: (i, 0)),
            pl.BlockSpec((D_IN, H), lambda i: (0, 0)),
            pl.BlockSpec((2, NB, 16), lambda i: (0, i, 0)),
        ],
        out_specs=pl.BlockSpec((1, NB, 128), lambda i: (0, i, 0)),
        out_shape=jax.ShapeDtypeStruct((1, N, 128), jnp.float32),
    )(x, W1, degp)


def _t2_body(s1_ref, t1_ref, degp_ref, b1_ref, out_ref):
    dv = _dv(degp_ref[...])
    u = dv * (s1_ref[0] + s1_ref[1] + t1_ref[0])
    h1 = jnp.maximum(u[:, :H] + b1_ref[...][0][None, :], 0.0) * dv
    out_ref[0] = jnp.concatenate([h1, jnp.zeros((NB, 128 - H), jnp.float32)], axis=1)


def _tc_t2(S1, t1, degp, b1):
    return pl.pallas_call(
        _t2_body,
        grid=(N // NB,),
        in_specs=[
            pl.BlockSpec((2, NB, 128), lambda i: (0, i, 0)),
            pl.BlockSpec((1, NB, 128), lambda i: (0, i, 0)),
            pl.BlockSpec((2, NB, 16), lambda i: (0, i, 0)),
            pl.BlockSpec((1, H), lambda i: (0, 0)),
        ],
        out_specs=pl.BlockSpec((1, NB, 128), lambda i: (0, i, 0)),
        out_shape=jax.ShapeDtypeStruct((1, N, 128), jnp.float32),
    )(S1, t1, degp, b1.reshape(1, H))


def _t3_body(s2_ref, t2_ref, degp_ref, w2_ref, b2_ref, out_ref):
    dv = _dv(degp_ref[...])
    u = dv * (s2_ref[0] + s2_ref[1] + t2_ref[0])
    h2 = jnp.dot(u[:, :H], w2_ref[...], preferred_element_type=jnp.float32)
    h2 = jnp.maximum(h2 + b2_ref[...][0][None, :], 0.0) * dv
    out_ref[0] = h2[:, :128]
    out_ref[1] = h2[:, 128:]


def _tc_t3(S2, t2, degp, W2, b2):
    return pl.pallas_call(
        _t3_body,
        grid=(N // NB,),
        in_specs=[
            pl.BlockSpec((2, NB, 128), lambda i: (0, i, 0)),
            pl.BlockSpec((1, NB, 128), lambda i: (0, i, 0)),
            pl.BlockSpec((2, NB, 16), lambda i: (0, i, 0)),
            pl.BlockSpec((H, 4 * H), lambda i: (0, 0)),
            pl.BlockSpec((1, 4 * H), lambda i: (0, 0)),
        ],
        out_specs=pl.BlockSpec((2, NB, 128), lambda i: (0, i, 0)),
        out_shape=jax.ShapeDtypeStruct((2, N, 128), jnp.float32),
    )(S2, t2, degp, W2, b2.reshape(1, 4 * H))


def _pool_body(s3_ref, t3_ref, degp_ref, w3_ref, b3_ref, batch_ref,
               psum_ref, pmax_ref):
    dv = _dv(degp_ref[...])
    acc = jnp.zeros((NB, 8 * H), jnp.float32)
    for c in range(2):
        u = dv * (s3_ref[c] + t3_ref[c])
        acc = acc + jnp.dot(u, w3_ref[c], preferred_element_type=jnp.float32)
    h3 = jnp.maximum(acc + b3_ref[...][0][None, :], 0.0)

    @pl.when(pl.program_id(0) == 0)
    def _():
        psum_ref[...] = jnp.zeros_like(psum_ref)
        pmax_ref[...] = jnp.full_like(pmax_ref, -jnp.inf)

    b = batch_ref[...]
    for g in range(G):
        mask = b == g
        psum_ref[g, :] += jnp.sum(jnp.where(mask, h3, 0.0), axis=0)
        pmax_ref[g, :] = jnp.maximum(
            pmax_ref[g, :], jnp.max(jnp.where(mask, h3, -jnp.inf), axis=0))


def _tc_pool(S3, t3, degp, W3r, b3, batch2d):
    return pl.pallas_call(
        _pool_body,
        grid=(N // NB,),
        in_specs=[
            pl.BlockSpec((2, NB, 128), lambda i: (0, i, 0)),
            pl.BlockSpec((2, NB, 128), lambda i: (0, i, 0)),
            pl.BlockSpec((2, NB, 16), lambda i: (0, i, 0)),
            pl.BlockSpec((2, 128, 8 * H), lambda i: (0, 0, 0)),
            pl.BlockSpec((1, 8 * H), lambda i: (0, 0)),
            pl.BlockSpec((NB, 1), lambda i: (i, 0)),
        ],
        out_specs=[
            pl.BlockSpec((G, 8 * H), lambda i: (0, 0)),
            pl.BlockSpec((G, 8 * H), lambda i: (0, 0)),
        ],
        out_shape=[
            jax.ShapeDtypeStruct((G, 8 * H), jnp.float32),
            jax.ShapeDtypeStruct((G, 8 * H), jnp.float32),
        ],
        compiler_params=pltpu.CompilerParams(
            dimension_semantics=("arbitrary",)),
    )(S3, t3, degp, W3r, b3.reshape(1, 8 * H), batch2d)


def _head_body(psum_ref, pmax_ref, batch_ref, wm1_ref, bm1_ref, wm2_ref,
               bm2_ref, out_ref):
    b = batch_ref[...]
    iota16 = lax.broadcasted_iota(jnp.int32, (G, 1), 0)
    inv = jnp.zeros((G, 1), jnp.float32)
    for g in range(G):
        cnt = jnp.sum(jnp.where(b == g, 1.0, 0.0))
        inv = inv + jnp.where(iota16 == g, 1.0 / jnp.maximum(cnt, 1.0), 0.0)
    mean = psum_ref[...] * inv
    hid = (jnp.dot(mean, wm1_ref[0], preferred_element_type=jnp.float32)
           + jnp.dot(pmax_ref[...], wm1_ref[1], preferred_element_type=jnp.float32)
           + bm1_ref[...][0][None, :])
    hid = jnp.maximum(hid, 0.0)
    out_ref[...] = (jnp.dot(hid, wm2_ref[...], preferred_element_type=jnp.float32)
                    + bm2_ref[...][0][None, :])


def _tc_head(psum, pmax, batch2d, Wm1r, bm1, Wm2, bm2):
    return pl.pallas_call(
        _head_body,
        out_shape=jax.ShapeDtypeStruct((G, NUM_CLASSES), jnp.float32),
    )(psum, pmax, batch2d, Wm1r, bm1.reshape(1, 8 * H), Wm2,
      bm2.reshape(1, NUM_CLASSES))


def kernel(x, edge_index, batch, W1, b1, W2, b2, W3, b3, Wm1, bm1, Wm2, bm2):
    pad = E_PAD - E
    pad_src = (jnp.arange(pad, dtype=jnp.int32) * 97) % N
    pad_dst = N_OUT + (jnp.arange(pad, dtype=jnp.int32) % 16)
    src2d = jnp.concatenate([edge_index[0], pad_src]).reshape(ROWS, EW)
    dst2d = jnp.concatenate([edge_index[1], pad_dst]).reshape(ROWS, EW)
    batch2d = batch.reshape(N, 1)
    ones16 = jnp.ones((EW, 16), jnp.float32)
    zeros16 = jnp.zeros((WCH, 16), jnp.float32)
    zeros128 = jnp.zeros((WCH, 128), jnp.float32)

    degp = _sc_degree(dst2d, ones16, zeros16)

    t1 = _tc_t1(x, W1, degp)
    S1 = _sc_propagate_split_edges(t1, src2d, dst2d, zeros128)
    t2 = _tc_t2(S1, t1, degp, b1)
    S2 = _sc_propagate_split_edges(t2, src2d, dst2d, zeros128)
    t3 = _tc_t3(S2, t2, degp, W2, b2)
    S3 = _sc_propagate_split_feats(t3, src2d, dst2d, zeros128)
    psum, pmax = _tc_pool(S3, t3, degp, W3.reshape(2, 128, 8 * H), b3, batch2d)
    return _tc_head(psum, pmax, batch2d, Wm1.reshape(2, 8 * H, 8 * H), bm1,
                    Wm2, bm2)
